# Initial kernel scaffold; baseline (speedup 1.0000x reference)
import jax
import jax.numpy as jnp
from jax.experimental import pallas as pl


def kernel(edge_index, x, x_t, x_i, x_n, rel_repr, edge_type, edge_norm, in_w, out_w, loop_w, w_rel, loop_rel, attn_w, bias, bn_gamma, bn_beta):
    raise NotImplementedError("write your pallas kernel here")


# speedup vs baseline: 7.3409x; 7.3409x over previous
import numpy as np
import jax
import jax.numpy as jnp
from jax import lax
from jax.experimental import pallas as pl
from jax.experimental.pallas import tpu as pltpu
from jax.experimental.pallas import tpu_sc as plsc

N = 10000
E = 320000
D = 128
F = 65
NRELP = 128
NC, NS = 2, 16
NW = NC * NS
EPW = E // NW
GCH = 80
NCH = EPW // GCH
NP = 10240
DRT = NP // 16
BE = 2000
NB = E // BE

def _dft_consts():
    k = np.arange(F)
    d = np.arange(D)
    ang_dk = 2.0 * np.pi * np.outer(d, k) / D
    ck = np.zeros((D, D), np.float32); ck[:, :F] = np.cos(ang_dk)
    sk = np.zeros((D, D), np.float32); sk[:, :F] = np.sin(ang_dk)
    w = np.full((F, 1), 2.0); w[0] = 1.0; w[F - 1] = 1.0
    ang_kn = 2.0 * np.pi * np.outer(k, d) / D
    c2 = np.zeros((D, D), np.float32); c2[:F, :] = w * np.cos(ang_kn) / D
    s2 = np.zeros((D, D), np.float32); s2[:F, :] = -w * np.sin(ang_kn) / D
    return ck, sk, c2, s2

_CK, _SK, _C2, _S2 = _dft_consts()


def _prep_body(x_ref, relp_ref, attn_ref, lrel_ref, lw_ref, inw_ref,
               outw_ref, wrel_ref, ck_ref, sk_ref, c2_ref, s2_ref,
               xc_ref, asrc_ref, adst_ref, arel_ref, btab_ref,
               gin_ref, gout_ref, loop_ref, relout_ref):
    x = x_ref[:]
    relp = relp_ref[:]
    ck = ck_ref[:]; sk = sk_ref[:]
    c2 = c2_ref[:]; s2 = s2_ref[:]
    f32 = jnp.float32
    xre = jnp.dot(x, ck, preferred_element_type=f32)
    xim = jnp.dot(x, sk, preferred_element_type=f32)
    xc_ref[:, :D] = xre
    xc_ref[:, D:] = xim
    attn = attn_ref[:]
    asrc_ref[:] = jnp.dot(x, attn[0:D, :], preferred_element_type=f32)
    adst_ref[:] = jnp.dot(x, attn[2 * D:3 * D, :], preferred_element_type=f32)
    arel_ref[:] = jnp.dot(relp, attn[D:2 * D, :], preferred_element_type=f32)
    btab_ref[:, :D] = jnp.dot(relp, ck, preferred_element_type=f32)
    btab_ref[:, D:] = -jnp.dot(relp, sk, preferred_element_type=f32)
    inw = inw_ref[:]; outw = outw_ref[:]
    gin_ref[0:D, :] = jnp.dot(c2, inw, preferred_element_type=f32)
    gin_ref[D:, :] = jnp.dot(s2, inw, preferred_element_type=f32)
    gout_ref[0:D, :] = jnp.dot(c2, outw, preferred_element_type=f32)
    gout_ref[D:, :] = jnp.dot(s2, outw, preferred_element_type=f32)
    lrel = lrel_ref[:]
    brl = jnp.dot(lrel, ck, preferred_element_type=f32)
    bil = -jnp.dot(lrel, sk, preferred_element_type=f32)
    zr = xre * brl - xim * bil
    zi = xre * bil + xim * brl
    lw = lw_ref[:]
    glc = jnp.dot(c2, lw, preferred_element_type=f32) * (1.0 / 3.0)
    gls = jnp.dot(s2, lw, preferred_element_type=f32) * (1.0 / 3.0)
    loop_ref[:] = (jnp.dot(zr, glc, preferred_element_type=f32)
                   + jnp.dot(zi, gls, preferred_element_type=f32))
    relout_ref[:] = jnp.dot(relp, wrel_ref[:], preferred_element_type=f32)


def _prep(x, relp, attn_w, loop_rel, loop_w, in_w, out_w, w_rel):
    f32 = jnp.float32
    return pl.pallas_call(
        _prep_body,
        out_shape=[
            jax.ShapeDtypeStruct((N, 2 * D), f32),
            jax.ShapeDtypeStruct((N, 1), f32),
            jax.ShapeDtypeStruct((N, 1), f32),
            jax.ShapeDtypeStruct((NRELP, 1), f32),
            jax.ShapeDtypeStruct((NRELP, 2 * D), f32),
            jax.ShapeDtypeStruct((2 * D, D), f32),
            jax.ShapeDtypeStruct((2 * D, D), f32),
            jax.ShapeDtypeStruct((N, D), f32),
            jax.ShapeDtypeStruct((NRELP, D), f32),
        ],
    )(x, relp, attn_w, loop_rel, loop_w, in_w, out_w, w_rel,
      jnp.asarray(_CK), jnp.asarray(_SK), jnp.asarray(_C2), jnp.asarray(_S2))


def _edge_body(xc_h, src_h, dst_h, et_h, asrc_h, adst_h, arel_h,
               arows_h, ex_h, denp_h,
               src_v, dst_v, et_v, asrc_v, adst_v, arel_v,
               ex_v, den_v, idx_v, rowbuf, shared_den, sem):
    c = lax.axis_index("c")
    s = lax.axis_index("s")
    wid = s * NC + c
    base = wid * EPW
    pltpu.sync_copy(src_h.at[pl.ds(base, EPW)], src_v)
    pltpu.sync_copy(dst_h.at[pl.ds(base, EPW)], dst_v)
    pltpu.sync_copy(et_h.at[pl.ds(base, EPW)], et_v)
    pltpu.sync_copy(asrc_h, asrc_v)
    pltpu.sync_copy(adst_h, adst_v)
    pltpu.sync_copy(arel_h, arel_v)

    def _zero(i, _):
        den_v[i, :] = jnp.zeros((16,), jnp.float32)
        return 0
    lax.fori_loop(0, DRT, _zero, 0)
    for j in range(DRT // 128):
        for i in range(8):
            idx_v[j, pl.ds(i * 16, 16)] = (lax.iota(jnp.int32, 16)
                                           + j * 128 + i * 16)

    @pl.when(s == 0)
    def _():
        pltpu.sync_copy(den_v, shared_den)
    plsc.subcore_barrier()

    def _scal(ei, _):
        off = ei * 16
        s16 = src_v[pl.ds(off, 16)]
        d16 = dst_v[pl.ds(off, 16)]
        e16 = et_v[pl.ds(off, 16)]
        lg = (plsc.load_gather(asrc_v, [s16])
              + plsc.load_gather(adst_v, [d16])
              + plsc.load_gather(arel_v, [e16]))
        lg = jnp.where(lg >= 0.0, lg, 0.01 * lg)
        ex16 = jnp.exp(lg)
        ex_v[pl.ds(off, 16)] = ex16
        row = jnp.right_shift(d16, 4)
        lane = jnp.bitwise_and(d16, 15)
        plsc.addupdate_scatter(den_v, [row, lane], ex16)
        return 0
    lax.fori_loop(0, EPW // 16, _scal, 0)
    pltpu.sync_copy(ex_v, ex_h.at[pl.ds(base, EPW)])

    for j in range(DRT // 128):
        pltpu.sync_copy(den_v.at[pl.ds(j * 128, 128)],
                        shared_den.at[idx_v.at[j]], add=True)
    plsc.subcore_barrier()
    rows = DRT // NS
    pltpu.sync_copy(shared_den.at[pl.ds(s * rows, rows)],
                    denp_h.at[pl.ds(c * DRT + s * rows, rows)])

    def _gath(ci, _):
        idx = src_v.at[pl.ds(ci * GCH, GCH)]
        pltpu.async_copy(xc_h.at[idx], rowbuf, sem).wait()
        pltpu.sync_copy(rowbuf, arows_h.at[pl.ds(base + ci * GCH, GCH)])
        return 0
    lax.fori_loop(0, NCH, _gath, 0)


def _edge_pass(xc, src, dst, et, a_src, a_dst, a_rel):
    f32 = jnp.float32
    mesh = plsc.VectorSubcoreMesh(core_axis_name="c", subcore_axis_name="s")
    kern = pl.kernel(
        _edge_body,
        out_type=[
            jax.ShapeDtypeStruct((E, 2 * D), f32),
            jax.ShapeDtypeStruct((E,), f32),
            jax.ShapeDtypeStruct((NC * DRT, 16), f32),
        ],
        mesh=mesh,
        compiler_params=pltpu.CompilerParams(needs_layout_passes=False, use_tc_tiling_on_sc=False),
        scratch_types=[
            pltpu.VMEM((EPW,), jnp.int32),
            pltpu.VMEM((EPW,), jnp.int32),
            pltpu.VMEM((EPW,), jnp.int32),
            pltpu.VMEM((N,), f32),
            pltpu.VMEM((N,), f32),
            pltpu.VMEM((NRELP,), f32),
            pltpu.VMEM((EPW,), f32),
            pltpu.VMEM((DRT, 16), f32),
            pltpu.VMEM((DRT // 128, 128), jnp.int32),
            pltpu.VMEM((GCH, 2 * D), f32),
            pltpu.VMEM_SHARED((DRT, 16), f32),
            pltpu.SemaphoreType.DMA,
        ],
    )
    return kern(xc, src, dst, et, a_src, a_dst, a_rel)


def _weight_body(denp_h, dst_h, ex_h, norm_h, w_h,
                 d0_v, d1_v, dst_v, ex_v, norm_v):
    c = lax.axis_index("c")
    s = lax.axis_index("s")
    wid = s * NC + c
    base = wid * EPW
    pltpu.sync_copy(denp_h.at[pl.ds(0, DRT)], d0_v)
    pltpu.sync_copy(denp_h.at[pl.ds(DRT, DRT)], d1_v)
    pltpu.sync_copy(dst_h.at[pl.ds(base, EPW)], dst_v)
    pltpu.sync_copy(ex_h.at[pl.ds(base, EPW)], ex_v)
    pltpu.sync_copy(norm_h.at[pl.ds(base, EPW)], norm_v)

    def _sum(i, _):
        d0_v[i, :] = d0_v[i, :] + d1_v[i, :]
        return 0
    lax.fori_loop(0, DRT, _sum, 0)

    def _w(ei, _):
        off = ei * 16
        d16 = dst_v[pl.ds(off, 16)]
        row = jnp.right_shift(d16, 4)
        lane = jnp.bitwise_and(d16, 15)
        den = plsc.load_gather(d0_v, [row, lane])
        ex16 = ex_v[pl.ds(off, 16)]
        nm16 = norm_v[pl.ds(off, 16)]
        ex_v[pl.ds(off, 16)] = ex16 * nm16 / den * (1.0 / 3.0)
        return 0
    lax.fori_loop(0, EPW // 16, _w, 0)
    pltpu.sync_copy(ex_v, w_h.at[pl.ds(base, EPW)])


def _weight_pass(denp, dst, ex, norm):
    f32 = jnp.float32
    mesh = plsc.VectorSubcoreMesh(core_axis_name="c", subcore_axis_name="s")
    kern = pl.kernel(
        _weight_body,
        out_type=jax.ShapeDtypeStruct((E,), f32),
        mesh=mesh,
        compiler_params=pltpu.CompilerParams(needs_layout_passes=False, use_tc_tiling_on_sc=False),
        scratch_types=[
            pltpu.VMEM((DRT, 16), f32),
            pltpu.VMEM((DRT, 16), f32),
            pltpu.VMEM((EPW,), jnp.int32),
            pltpu.VMEM((EPW,), f32),
            pltpu.VMEM((EPW,), f32),
        ],
    )
    return kern(denp, dst, ex, norm)


def _msg_body(arows_ref, et_ref, w_ref, btab_ref, gin_ref, gout_ref,
              out_ref):
    f32 = jnp.float32
    i = pl.program_id(0)
    a = arows_ref[:]
    a_re = a[:, :D]
    a_im = a[:, D:]
    et = et_ref[:]
    lanes = lax.broadcasted_iota(jnp.int32, (BE, NRELP), 1)
    oh = (et == lanes).astype(f32)
    b = jnp.dot(oh, btab_ref[:], preferred_element_type=f32)
    b_re = b[:, :D]
    b_im = b[:, D:]
    zr = a_re * b_re - a_im * b_im
    zi = a_re * b_im + a_im * b_re
    g = jnp.where(i < NB // 2, gin_ref[:], gout_ref[:])
    mp = (jnp.dot(zr, g[:D, :], preferred_element_type=f32)
          + jnp.dot(zi, g[D:, :], preferred_element_type=f32))
    m = jnp.max(mp, axis=1, keepdims=True)
    p = jnp.exp(mp - m)
    ssum = jnp.sum(p, axis=1, keepdims=True)
    out_ref[:] = p * (w_ref[:] / ssum)


def _messages(arows, et2, w2, btab, gin, gout):
    f32 = jnp.float32
    return pl.pallas_call(
        _msg_body,
        grid=(NB,),
        in_specs=[
            pl.BlockSpec((BE, 2 * D), lambda i: (i, 0)),
            pl.BlockSpec((BE, 1), lambda i: (i, 0)),
            pl.BlockSpec((BE, 1), lambda i: (i, 0)),
            pl.BlockSpec((NRELP, 2 * D), lambda i: (0, 0)),
            pl.BlockSpec((2 * D, D), lambda i: (0, 0)),
            pl.BlockSpec((2 * D, D), lambda i: (0, 0)),
        ],
        out_specs=pl.BlockSpec((BE, D), lambda i: (i, 0)),
        out_shape=jax.ShapeDtypeStruct((E, D), f32),
    )(arows, et2, w2, btab, gin, gout)


def _scatter_body(msg_h, dst3_h, hagg_h, dst_v, rowbuf, shared_h, sem):
    c = lax.axis_index("c")
    s = lax.axis_index("s")
    wid = s * NC + c
    base = wid * EPW
    pltpu.sync_copy(dst3_h.at[wid], dst_v)

    def _zrow(r, _):
        for j in range(D // 16):
            rowbuf[r, pl.ds(j * 16, 16)] = jnp.zeros((16,), jnp.float32)
        return 0
    lax.fori_loop(0, GCH, _zrow, 0)
    for t in range(NP // 16 // GCH):
        pltpu.sync_copy(rowbuf,
                        shared_h.at[pl.ds(s * (NP // 16) + t * GCH, GCH)])
    plsc.subcore_barrier()

    def _chunk(ci, _):
        pltpu.sync_copy(msg_h.at[pl.ds(base + ci * GCH, GCH)], rowbuf)
        pltpu.sync_copy(rowbuf, shared_h.at[dst_v.at[ci]], add=True)
        return 0
    lax.fori_loop(0, NCH, _chunk, 0)
    plsc.subcore_barrier()
    pltpu.sync_copy(shared_h.at[pl.ds(s * (NP // 16), NP // 16)],
                    hagg_h.at[pl.ds(c * NP + s * (NP // 16), NP // 16)])


def _scatter(msg, dst3):
    f32 = jnp.float32
    mesh = plsc.VectorSubcoreMesh(core_axis_name="c", subcore_axis_name="s")
    kern = pl.kernel(
        _scatter_body,
        out_type=jax.ShapeDtypeStruct((NC * NP, D), f32),
        mesh=mesh,
        compiler_params=pltpu.CompilerParams(needs_layout_passes=False, use_tc_tiling_on_sc=False),
        scratch_types=[
            pltpu.VMEM((NCH, GCH), jnp.int32),
            pltpu.VMEM((GCH, D), f32),
            pltpu.VMEM_SHARED((NP, D), f32),
            pltpu.SemaphoreType.DMA,
        ],
    )
    return kern(msg, dst3)


def _final_body(hagg_ref, loop_ref, bias_ref, gam_ref, bet_ref, out_ref):
    h = (hagg_ref[0:N, :] + hagg_ref[NP:NP + N, :]
         + loop_ref[:] + bias_ref[:])
    mu = jnp.mean(h, axis=0, keepdims=True)
    d = h - mu
    var = jnp.mean(d * d, axis=0, keepdims=True)
    out_ref[:] = d * (gam_ref[:] * lax.rsqrt(var + 1e-5)) + bet_ref[:]


def _finalize(hagg, loop_t, bias, gamma, beta):
    f32 = jnp.float32
    return pl.pallas_call(
        _final_body,
        out_shape=jax.ShapeDtypeStruct((N, D), f32),
    )(hagg, loop_t, bias, gamma, beta)


def kernel(edge_index, x, x_t, x_i, x_n, rel_repr, edge_type, edge_norm,
           in_w, out_w, loop_w, w_rel, loop_rel, attn_w, bias,
           bn_gamma, bn_beta):
    src = edge_index[0]
    dst = edge_index[1]
    relp = jnp.pad(rel_repr, ((0, NRELP - rel_repr.shape[0]), (0, 0)))

    (xc, a_src, a_dst, a_rel, btab, gin, gout, loop_t, relout_p) = _prep(
        x, relp, attn_w, loop_rel, loop_w, in_w, out_w, w_rel)

    arows, ex, denp = _edge_pass(
        xc, src, dst, edge_type,
        a_src.reshape(N), a_dst.reshape(N), a_rel.reshape(NRELP))

    weight = _weight_pass(denp, dst, ex, edge_norm)

    msg = _messages(arows, edge_type.reshape(E, 1),
                    weight.reshape(E, 1), btab, gin, gout)

    hagg = _scatter(msg, dst.reshape(NW, NCH, GCH))

    out = _finalize(hagg, loop_t, bias.reshape(1, D),
                    bn_gamma.reshape(1, D), bn_beta.reshape(1, D))
    return (out, relout_p[:rel_repr.shape[0], :])


# speedup vs baseline: 10.7542x; 1.4650x over previous
import numpy as np
import jax
import jax.numpy as jnp
from jax import lax
from jax.experimental import pallas as pl
from jax.experimental.pallas import tpu as pltpu
from jax.experimental.pallas import tpu_sc as plsc

N = 10000
E = 320000
D = 128
F = 65
NRELP = 128
NC, NS = 2, 16
NW = NC * NS
EPW = E // NW
GCH = 40
NCH = EPW // GCH
NP = 10240
DRT = NP // 16
BE = 4000
NB = E // BE

def _dft_consts():
    k = np.arange(F)
    d = np.arange(D)
    binmap = np.array([l if l < F else l - 64 for l in range(D)])
    is_re = np.arange(D) < F
    ang = 2.0 * np.pi * np.outer(d, binmap) / D
    cs = np.where(is_re[None, :], np.cos(ang), np.sin(ang))
    cv1 = np.cos(ang)
    cv2 = -np.sin(ang)
    w = np.full((F, 1), 2.0); w[0] = 1.0; w[F - 1] = 1.0
    ang_kn = 2.0 * np.pi * np.outer(k, d) / D
    c2 = w * np.cos(ang_kn) / D
    s2 = -w * np.sin(ang_kn) / D
    ca = np.where(is_re[:, None], c2[binmap, :], s2[binmap, :])
    cb = np.where(is_re[:, None], s2[binmap, :], -c2[binmap, :])
    f32 = np.float32
    return cs.astype(f32), cv1.astype(f32), cv2.astype(f32), \
        ca.astype(f32), cb.astype(f32)

_CS, _CV1, _CV2, _CA, _CB = _dft_consts()


def _prep_body(x_ref, relp_ref, attn_ref, lrel_ref, lw_ref, inw_ref,
               outw_ref, wrel_ref, cs_ref, cv1_ref, cv2_ref, ca_ref, cb_ref,
               xu_ref, asrc_ref, adst_ref, arel_ref, vtab_ref,
               a2_ref, b2_ref, loop_ref, relout_ref):
    x = x_ref[:]
    relp = relp_ref[:]
    cs = cs_ref[:]; cv1 = cv1_ref[:]; cv2 = cv2_ref[:]
    ca = ca_ref[:]; cb = cb_ref[:]
    f32 = jnp.float32
    xu = jnp.dot(x, cs, preferred_element_type=f32)
    xu_ref[:] = xu
    attn = attn_ref[:]
    asrc_ref[:] = jnp.dot(x, attn[0:D, :], preferred_element_type=f32)
    adst_ref[:] = jnp.dot(x, attn[2 * D:3 * D, :], preferred_element_type=f32)
    arel_ref[:] = jnp.dot(relp, attn[D:2 * D, :], preferred_element_type=f32)
    vtab_ref[:, :D] = jnp.dot(relp, cv1, preferred_element_type=f32)
    vtab_ref[:, D:] = jnp.dot(relp, cv2, preferred_element_type=f32)
    inw = inw_ref[:]; outw = outw_ref[:]
    a2_ref[0] = jnp.dot(ca, inw, preferred_element_type=f32)
    a2_ref[1] = jnp.dot(ca, outw, preferred_element_type=f32)
    b2_ref[0] = jnp.dot(cb, inw, preferred_element_type=f32)
    b2_ref[1] = jnp.dot(cb, outw, preferred_element_type=f32)
    lrel = lrel_ref[:]
    v1l = jnp.dot(lrel, cv1, preferred_element_type=f32)
    v2l = jnp.dot(lrel, cv2, preferred_element_type=f32)
    lw = lw_ref[:]
    al = jnp.dot(ca, lw, preferred_element_type=f32) * (1.0 / 3.0)
    bl = jnp.dot(cb, lw, preferred_element_type=f32) * (1.0 / 3.0)
    loop_ref[:] = (jnp.dot(xu * v1l, al, preferred_element_type=f32)
                   + jnp.dot(xu * v2l, bl, preferred_element_type=f32))
    relout_ref[:] = jnp.dot(relp, wrel_ref[:], preferred_element_type=f32)


def _prep(x, relp, attn_w, loop_rel, loop_w, in_w, out_w, w_rel):
    f32 = jnp.float32
    return pl.pallas_call(
        _prep_body,
        out_shape=[
            jax.ShapeDtypeStruct((N, D), f32),
            jax.ShapeDtypeStruct((N, 1), f32),
            jax.ShapeDtypeStruct((N, 1), f32),
            jax.ShapeDtypeStruct((NRELP, 1), f32),
            jax.ShapeDtypeStruct((NRELP, 2 * D), f32),
            jax.ShapeDtypeStruct((2, D, D), f32),
            jax.ShapeDtypeStruct((2, D, D), f32),
            jax.ShapeDtypeStruct((N, D), f32),
            jax.ShapeDtypeStruct((NRELP, D), f32),
        ],
    )(x, relp, attn_w, loop_rel, loop_w, in_w, out_w, w_rel,
      jnp.asarray(_CS), jnp.asarray(_CV1), jnp.asarray(_CV2),
      jnp.asarray(_CA), jnp.asarray(_CB))


def _edge_body(xu_h, src_h, dst_h, et_h, asrc_h, adst_h, arel_h,
               arows_h, ex_h, denp_h,
               src_v, dst_v, et_v, asrc_v, adst_v, arel_v,
               ex_v, den_v, idx_v, rb0, rb1, shared_den, sem0, sem1):
    c = lax.axis_index("c")
    s = lax.axis_index("s")
    wid = s * NC + c
    base = wid * EPW
    pltpu.sync_copy(src_h.at[pl.ds(base, EPW)], src_v)
    pltpu.sync_copy(dst_h.at[pl.ds(base, EPW)], dst_v)
    pltpu.sync_copy(et_h.at[pl.ds(base, EPW)], et_v)
    pltpu.sync_copy(asrc_h, asrc_v)
    pltpu.sync_copy(adst_h, adst_v)
    pltpu.sync_copy(arel_h, arel_v)

    def _zero(i, _):
        den_v[i, :] = jnp.zeros((16,), jnp.float32)
        return 0
    lax.fori_loop(0, DRT, _zero, 0)
    for j in range(DRT // 128):
        for i in range(8):
            idx_v[j, pl.ds(i * 16, 16)] = (lax.iota(jnp.int32, 16)
                                           + j * 128 + i * 16)

    @pl.when(s == 0)
    def _():
        pltpu.sync_copy(den_v, shared_den)
    plsc.subcore_barrier()

    def _scal(ei, _):
        off = ei * 16
        s16 = src_v[pl.ds(off, 16)]
        d16 = dst_v[pl.ds(off, 16)]
        e16 = et_v[pl.ds(off, 16)]
        lg = (plsc.load_gather(asrc_v, [s16])
              + plsc.load_gather(adst_v, [d16])
              + plsc.load_gather(arel_v, [e16]))
        lg = jnp.where(lg >= 0.0, lg, 0.01 * lg)
        ex16 = jnp.exp(lg)
        ex_v[pl.ds(off, 16)] = ex16
        row = jnp.right_shift(d16, 4)
        lane = jnp.bitwise_and(d16, 15)
        plsc.addupdate_scatter(den_v, [row, lane], ex16)
        return 0
    lax.fori_loop(0, EPW // 16, _scal, 0)
    pltpu.sync_copy(ex_v, ex_h.at[pl.ds(base, EPW)])

    for j in range(DRT // 128):
        pltpu.sync_copy(den_v.at[pl.ds(j * 128, 128)],
                        shared_den.at[idx_v.at[j]], add=True)
    plsc.subcore_barrier()
    rows = DRT // NS
    pltpu.sync_copy(shared_den.at[pl.ds(s * rows, rows)],
                    denp_h.at[pl.ds(c * DRT + s * rows, rows)])

    def _start(ci, rb, sem):
        pltpu.async_copy(xu_h.at[src_v.at[pl.ds(ci * GCH, GCH)]], rb, sem)

    def _wait(rb, sem):
        pltpu.make_async_copy(xu_h.at[pl.ds(0, GCH)], rb, sem).wait()

    _start(0, rb0, sem0)

    def _gath(i, _):
        _start(2 * i + 1, rb1, sem1)
        _wait(rb0, sem0)
        pltpu.sync_copy(rb0, arows_h.at[pl.ds(base + (2 * i) * GCH, GCH)])

        @pl.when(i < NCH // 2 - 1)
        def _():
            _start(2 * i + 2, rb0, sem0)
        _wait(rb1, sem1)
        pltpu.sync_copy(rb1, arows_h.at[pl.ds(base + (2 * i + 1) * GCH, GCH)])
        return 0
    lax.fori_loop(0, NCH // 2, _gath, 0)


def _edge_pass(xu, src, dst, et, a_src, a_dst, a_rel):
    f32 = jnp.float32
    mesh = plsc.VectorSubcoreMesh(core_axis_name="c", subcore_axis_name="s")
    kern = pl.kernel(
        _edge_body,
        out_type=[
            jax.ShapeDtypeStruct((E, D), f32),
            jax.ShapeDtypeStruct((E,), f32),
            jax.ShapeDtypeStruct((NC * DRT, 16), f32),
        ],
        mesh=mesh,
        compiler_params=pltpu.CompilerParams(
            needs_layout_passes=False, use_tc_tiling_on_sc=False),
        scratch_types=[
            pltpu.VMEM((EPW,), jnp.int32),
            pltpu.VMEM((EPW,), jnp.int32),
            pltpu.VMEM((EPW,), jnp.int32),
            pltpu.VMEM((N,), f32),
            pltpu.VMEM((N,), f32),
            pltpu.VMEM((NRELP,), f32),
            pltpu.VMEM((EPW,), f32),
            pltpu.VMEM((DRT, 16), f32),
            pltpu.VMEM((DRT // 128, 128), jnp.int32),
            pltpu.VMEM((GCH, D), f32),
            pltpu.VMEM((GCH, D), f32),
            pltpu.VMEM_SHARED((DRT, 16), f32),
            pltpu.SemaphoreType.DMA,
            pltpu.SemaphoreType.DMA,
        ],
    )
    return kern(xu, src, dst, et, a_src, a_dst, a_rel)


def _weight_body(denp_h, dst_h, ex_h, norm_h, w_h,
                 d0_v, d1_v, dst_v, ex_v, norm_v):
    c = lax.axis_index("c")
    s = lax.axis_index("s")
    wid = s * NC + c
    base = wid * EPW
    pltpu.sync_copy(denp_h.at[pl.ds(0, DRT)], d0_v)
    pltpu.sync_copy(denp_h.at[pl.ds(DRT, DRT)], d1_v)
    pltpu.sync_copy(dst_h.at[pl.ds(base, EPW)], dst_v)
    pltpu.sync_copy(ex_h.at[pl.ds(base, EPW)], ex_v)
    pltpu.sync_copy(norm_h.at[pl.ds(base, EPW)], norm_v)

    def _sum(i, _):
        d0_v[i, :] = d0_v[i, :] + d1_v[i, :]
        return 0
    lax.fori_loop(0, DRT, _sum, 0)

    def _w(ei, _):
        off = ei * 16
        d16 = dst_v[pl.ds(off, 16)]
        row = jnp.right_shift(d16, 4)
        lane = jnp.bitwise_and(d16, 15)
        den = plsc.load_gather(d0_v, [row, lane])
        ex16 = ex_v[pl.ds(off, 16)]
        nm16 = norm_v[pl.ds(off, 16)]
        ex_v[pl.ds(off, 16)] = ex16 * nm16 / den * (1.0 / 3.0)
        return 0
    lax.fori_loop(0, EPW // 16, _w, 0)
    pltpu.sync_copy(ex_v, w_h.at[pl.ds(base, EPW)])


def _weight_pass(denp, dst, ex, norm):
    f32 = jnp.float32
    mesh = plsc.VectorSubcoreMesh(core_axis_name="c", subcore_axis_name="s")
    kern = pl.kernel(
        _weight_body,
        out_type=jax.ShapeDtypeStruct((E,), f32),
        mesh=mesh,
        compiler_params=pltpu.CompilerParams(
            needs_layout_passes=False, use_tc_tiling_on_sc=False),
        scratch_types=[
            pltpu.VMEM((DRT, 16), f32),
            pltpu.VMEM((DRT, 16), f32),
            pltpu.VMEM((EPW,), jnp.int32),
            pltpu.VMEM((EPW,), f32),
            pltpu.VMEM((EPW,), f32),
        ],
    )
    return kern(denp, dst, ex, norm)


def _msg_body(u_ref, et_ref, w_ref, vtab_ref, a_ref, b_ref, out_ref):
    f32 = jnp.float32
    u = u_ref[:]
    et = et_ref[:]
    lanes = lax.broadcasted_iota(jnp.int32, (BE, NRELP), 1)
    oh = (et == lanes).astype(f32)
    v = jnp.dot(oh, vtab_ref[:], preferred_element_type=f32)
    w1 = u * v[:, :D]
    w2 = u * v[:, D:]
    mp = (jnp.dot(w1, a_ref[0], preferred_element_type=f32)
          + jnp.dot(w2, b_ref[0], preferred_element_type=f32))
    m = jnp.max(mp, axis=1, keepdims=True)
    p = jnp.exp(mp - m)
    ssum = jnp.sum(p, axis=1, keepdims=True)
    out_ref[:] = p * (w_ref[:] / ssum)


def _messages(arows, et2, w2, vtab, a2, b2):
    f32 = jnp.float32
    return pl.pallas_call(
        _msg_body,
        grid=(NB,),
        in_specs=[
            pl.BlockSpec((BE, D), lambda i: (i, 0)),
            pl.BlockSpec((BE, 1), lambda i: (i, 0)),
            pl.BlockSpec((BE, 1), lambda i: (i, 0)),
            pl.BlockSpec((NRELP, 2 * D), lambda i: (0, 0)),
            pl.BlockSpec((1, D, D), lambda i: (i // (NB // 2), 0, 0)),
            pl.BlockSpec((1, D, D), lambda i: (i // (NB // 2), 0, 0)),
        ],
        out_specs=pl.BlockSpec((BE, D), lambda i: (i, 0)),
        out_shape=jax.ShapeDtypeStruct((E, D), f32),
    )(arows, et2, w2, vtab, a2, b2)


def _scatter_body(msg_h, dst3_h, hagg_h, dst_v, rb0, rb1, shared_h,
                  sem0, sem1):
    c = lax.axis_index("c")
    s = lax.axis_index("s")
    wid = s * NC + c
    base = wid * EPW
    pltpu.sync_copy(dst3_h.at[wid], dst_v)

    def _zrow(r, _):
        for j in range(D // 16):
            rb0[r, pl.ds(j * 16, 16)] = jnp.zeros((16,), jnp.float32)
        return 0
    lax.fori_loop(0, GCH, _zrow, 0)
    for t in range(NP // NS // GCH):
        pltpu.sync_copy(rb0,
                        shared_h.at[pl.ds(s * (NP // NS) + t * GCH, GCH)])
    plsc.subcore_barrier()

    def _start(ci, rb, sem):
        pltpu.async_copy(msg_h.at[pl.ds(base + ci * GCH, GCH)], rb, sem)

    def _wait(rb, sem):
        pltpu.make_async_copy(msg_h.at[pl.ds(0, GCH)], rb, sem).wait()

    _start(0, rb0, sem0)

    def _chunk(i, _):
        _start(2 * i + 1, rb1, sem1)
        _wait(rb0, sem0)
        pltpu.sync_copy(rb0, shared_h.at[dst_v.at[2 * i]], add=True)

        @pl.when(i < NCH // 2 - 1)
        def _():
            _start(2 * i + 2, rb0, sem0)
        _wait(rb1, sem1)
        pltpu.sync_copy(rb1, shared_h.at[dst_v.at[2 * i + 1]], add=True)
        return 0
    lax.fori_loop(0, NCH // 2, _chunk, 0)
    plsc.subcore_barrier()
    pltpu.sync_copy(shared_h.at[pl.ds(s * (NP // NS), NP // NS)],
                    hagg_h.at[pl.ds(c * NP + s * (NP // NS), NP // NS)])


def _scatter(msg, dst3):
    f32 = jnp.float32
    mesh = plsc.VectorSubcoreMesh(core_axis_name="c", subcore_axis_name="s")
    kern = pl.kernel(
        _scatter_body,
        out_type=jax.ShapeDtypeStruct((NC * NP, D), f32),
        mesh=mesh,
        compiler_params=pltpu.CompilerParams(
            needs_layout_passes=False, use_tc_tiling_on_sc=False),
        scratch_types=[
            pltpu.VMEM((NCH, GCH), jnp.int32),
            pltpu.VMEM((GCH, D), f32),
            pltpu.VMEM((GCH, D), f32),
            pltpu.VMEM_SHARED((NP, D), f32),
            pltpu.SemaphoreType.DMA,
            pltpu.SemaphoreType.DMA,
        ],
    )
    return kern(msg, dst3)


def _final_body(hagg_ref, loop_ref, bias_ref, gam_ref, bet_ref, out_ref):
    h = (hagg_ref[0:N, :] + hagg_ref[NP:NP + N, :]
         + loop_ref[:] + bias_ref[:])
    mu = jnp.mean(h, axis=0, keepdims=True)
    d = h - mu
    var = jnp.mean(d * d, axis=0, keepdims=True)
    out_ref[:] = d * (gam_ref[:] * lax.rsqrt(var + 1e-5)) + bet_ref[:]


def _finalize(hagg, loop_t, bias, gamma, beta):
    f32 = jnp.float32
    return pl.pallas_call(
        _final_body,
        out_shape=jax.ShapeDtypeStruct((N, D), f32),
    )(hagg, loop_t, bias, gamma, beta)


def kernel(edge_index, x, x_t, x_i, x_n, rel_repr, edge_type, edge_norm,
           in_w, out_w, loop_w, w_rel, loop_rel, attn_w, bias,
           bn_gamma, bn_beta):
    src = edge_index[0]
    dst = edge_index[1]
    relp = jnp.pad(rel_repr, ((0, NRELP - rel_repr.shape[0]), (0, 0)))

    (xu, a_src, a_dst, a_rel, vtab, a2, b2, loop_t, relout_p) = _prep(
        x, relp, attn_w, loop_rel, loop_w, in_w, out_w, w_rel)

    arows, ex, denp = _edge_pass(
        xu, src, dst, edge_type,
        a_src.reshape(N), a_dst.reshape(N), a_rel.reshape(NRELP))

    weight = _weight_pass(denp, dst, ex, edge_norm)

    msg = _messages(arows, edge_type.reshape(E, 1),
                    weight.reshape(E, 1), vtab, a2, b2)

    hagg = _scatter(msg, dst.reshape(NW, NCH, GCH))

    out = _finalize(hagg, loop_t, bias.reshape(1, D),
                    bn_gamma.reshape(1, D), bn_beta.reshape(1, D))
    return (out, relout_p[:rel_repr.shape[0], :])


# speedup vs baseline: 19.4447x; 1.8081x over previous
import numpy as np
import jax
import jax.numpy as jnp
from jax import lax
from jax.experimental import pallas as pl
from jax.experimental.pallas import tpu as pltpu
from jax.experimental.pallas import tpu_sc as plsc

N = 10000
E = 320000
D = 128
F = 65
NRELP = 128
NC, NS = 2, 16
NW = NC * NS
EPW = E // NW
GCH = 40
NCH = EPW // GCH
NP = 10240
DRT = NP // D
BE = 3200
NB = E // BE

def _dft_consts():
    k = np.arange(F)
    d = np.arange(D)
    binmap = np.array([l if l < F else l - 64 for l in range(D)])
    is_re = np.arange(D) < F
    ang = 2.0 * np.pi * np.outer(d, binmap) / D
    cs = np.where(is_re[None, :], np.cos(ang), np.sin(ang))
    cv1 = np.cos(ang)
    cv2 = -np.sin(ang)
    w = np.full((F, 1), 2.0); w[0] = 1.0; w[F - 1] = 1.0
    ang_kn = 2.0 * np.pi * np.outer(k, d) / D
    c2 = w * np.cos(ang_kn) / D
    s2 = -w * np.sin(ang_kn) / D
    ca = np.where(is_re[:, None], c2[binmap, :], s2[binmap, :])
    cb = np.where(is_re[:, None], s2[binmap, :], -c2[binmap, :])
    f32 = np.float32
    return cs.astype(f32), cv1.astype(f32), cv2.astype(f32), \
        ca.astype(f32), cb.astype(f32)

_CS, _CV1, _CV2, _CA, _CB = _dft_consts()


def _prep_body(x_ref, relp_ref, attn_ref, lrel_ref, lw_ref, inw_ref,
               outw_ref, wrel_ref, cs_ref, cv1_ref, cv2_ref, ca_ref, cb_ref,
               xu_ref, asrc_ref, adst_ref, arel_ref, vhi_ref, vlo_ref,
               ab_ref, loop_ref, relout_ref):
    x = x_ref[:]
    relp = relp_ref[:]
    cs = cs_ref[:]; cv1 = cv1_ref[:]; cv2 = cv2_ref[:]
    ca = ca_ref[:]; cb = cb_ref[:]
    f32 = jnp.float32
    xu = jnp.dot(x, cs, preferred_element_type=f32)
    xu_ref[:] = xu
    attn = attn_ref[:]
    asrc_ref[:] = jnp.dot(x, attn[0:D, :], preferred_element_type=f32)
    adst_ref[:] = jnp.dot(x, attn[2 * D:3 * D, :], preferred_element_type=f32)
    arel_ref[:] = jnp.dot(relp, attn[D:2 * D, :], preferred_element_type=f32)
    v1t = jnp.dot(relp, cv1, preferred_element_type=f32)
    v2t = jnp.dot(relp, cv2, preferred_element_type=f32)
    vtab = jnp.concatenate([v1t, v2t], axis=1)
    vhi = vtab.astype(jnp.bfloat16)
    vhi_ref[:] = vhi
    vlo_ref[:] = (vtab - vhi.astype(f32)).astype(jnp.bfloat16)
    inw = inw_ref[:]; outw = outw_ref[:]
    ab_ref[0, :D, :] = jnp.dot(ca, inw, preferred_element_type=f32)
    ab_ref[0, D:, :] = jnp.dot(cb, inw, preferred_element_type=f32)
    ab_ref[1, :D, :] = jnp.dot(ca, outw, preferred_element_type=f32)
    ab_ref[1, D:, :] = jnp.dot(cb, outw, preferred_element_type=f32)
    lrel = lrel_ref[:]
    v1l = jnp.dot(lrel, cv1, preferred_element_type=f32)
    v2l = jnp.dot(lrel, cv2, preferred_element_type=f32)
    lw = lw_ref[:]
    al = jnp.dot(ca, lw, preferred_element_type=f32) * (1.0 / 3.0)
    bl = jnp.dot(cb, lw, preferred_element_type=f32) * (1.0 / 3.0)
    loop_ref[:] = (jnp.dot(xu * v1l, al, preferred_element_type=f32)
                   + jnp.dot(xu * v2l, bl, preferred_element_type=f32))
    relout_ref[:] = jnp.dot(relp, wrel_ref[:], preferred_element_type=f32)


def _prep(x, relp, attn_w, loop_rel, loop_w, in_w, out_w, w_rel):
    f32 = jnp.float32
    return pl.pallas_call(
        _prep_body,
        out_shape=[
            jax.ShapeDtypeStruct((N, D), f32),
            jax.ShapeDtypeStruct((N, 1), f32),
            jax.ShapeDtypeStruct((N, 1), f32),
            jax.ShapeDtypeStruct((NRELP, 1), f32),
            jax.ShapeDtypeStruct((NRELP, 2 * D), jnp.bfloat16),
            jax.ShapeDtypeStruct((NRELP, 2 * D), jnp.bfloat16),
            jax.ShapeDtypeStruct((2, 2 * D, D), f32),
            jax.ShapeDtypeStruct((N, D), f32),
            jax.ShapeDtypeStruct((NRELP, D), f32),
        ],
    )(x, relp, attn_w, loop_rel, loop_w, in_w, out_w, w_rel,
      jnp.asarray(_CS), jnp.asarray(_CV1), jnp.asarray(_CV2),
      jnp.asarray(_CA), jnp.asarray(_CB))


def _edge_body(xu_h, src_h, dst_h, et_h, asrc_h, adst_h, arel_h,
               arows_h, ex_h, denp_h,
               src_v, dst_v, et_v, asrc_v, adst_v, arel_v,
               ex_v, den_v, idx_v, rb0, rb1, shared_den, sem0, sem1):
    c = lax.axis_index("c")
    s = lax.axis_index("s")
    wid = s * NC + c
    base = wid * EPW
    pltpu.sync_copy(src_h.at[pl.ds(base, EPW)], src_v)
    pltpu.sync_copy(dst_h.at[pl.ds(base, EPW)], dst_v)
    pltpu.sync_copy(et_h.at[pl.ds(base, EPW)], et_v)
    pltpu.sync_copy(asrc_h, asrc_v)
    pltpu.sync_copy(adst_h, adst_v)
    pltpu.sync_copy(arel_h, arel_v)

    def _zero(i, _):
        for j in range(D // 16):
            den_v[i, pl.ds(j * 16, 16)] = jnp.zeros((16,), jnp.float32)
        return 0
    lax.fori_loop(0, DRT, _zero, 0)
    for i in range(DRT // 16):
        idx_v[0, pl.ds(i * 16, 16)] = lax.iota(jnp.int32, 16) + i * 16

    @pl.when(s == 0)
    def _():
        pltpu.sync_copy(den_v, shared_den)
    plsc.subcore_barrier()

    def _scal(ei, _):
        off = ei * 16
        s16 = src_v[pl.ds(off, 16)]
        d16 = dst_v[pl.ds(off, 16)]
        e16 = et_v[pl.ds(off, 16)]
        lg = (plsc.load_gather(asrc_v, [s16])
              + plsc.load_gather(adst_v, [d16])
              + plsc.load_gather(arel_v, [e16]))
        lg = jnp.where(lg >= 0.0, lg, 0.01 * lg)
        ex16 = jnp.exp(lg)
        ex_v[pl.ds(off, 16)] = ex16
        row = jnp.right_shift(d16, 7)
        lane = jnp.bitwise_and(d16, 127)
        plsc.addupdate_scatter(den_v, [row, lane], ex16)
        return 0
    lax.fori_loop(0, EPW // 16, _scal, 0)
    pltpu.sync_copy(ex_v, ex_h.at[pl.ds(base, EPW)])

    pltpu.sync_copy(den_v, shared_den.at[idx_v.at[0]], add=True)
    plsc.subcore_barrier()

    @pl.when(s < DRT // 8)
    def _():
        pltpu.sync_copy(shared_den.at[pl.ds(s * 8, 8)],
                        denp_h.at[pl.ds(c * DRT + s * 8, 8)])

    def _start(ci, rb, sem):
        pltpu.async_copy(xu_h.at[src_v.at[pl.ds(ci * GCH, GCH)]], rb, sem)

    def _wait(rb, sem):
        pltpu.make_async_copy(xu_h.at[pl.ds(0, GCH)], rb, sem).wait()

    _start(0, rb0, sem0)

    def _gath(i, _):
        _start(2 * i + 1, rb1, sem1)
        _wait(rb0, sem0)
        pltpu.sync_copy(rb0, arows_h.at[pl.ds(base + (2 * i) * GCH, GCH)])

        @pl.when(i < NCH // 2 - 1)
        def _():
            _start(2 * i + 2, rb0, sem0)
        _wait(rb1, sem1)
        pltpu.sync_copy(rb1, arows_h.at[pl.ds(base + (2 * i + 1) * GCH, GCH)])
        return 0
    lax.fori_loop(0, NCH // 2, _gath, 0)


def _edge_pass(xu, src, dst, et, a_src, a_dst, a_rel):
    f32 = jnp.float32
    mesh = plsc.VectorSubcoreMesh(core_axis_name="c", subcore_axis_name="s")
    kern = pl.kernel(
        _edge_body,
        out_type=[
            jax.ShapeDtypeStruct((E, D), f32),
            jax.ShapeDtypeStruct((E,), f32),
            jax.ShapeDtypeStruct((NC * DRT, D), f32),
        ],
        mesh=mesh,
        compiler_params=pltpu.CompilerParams(
            needs_layout_passes=False, use_tc_tiling_on_sc=True),
        scratch_types=[
            pltpu.VMEM((EPW,), jnp.int32),
            pltpu.VMEM((EPW,), jnp.int32),
            pltpu.VMEM((EPW,), jnp.int32),
            pltpu.VMEM((N,), f32),
            pltpu.VMEM((N,), f32),
            pltpu.VMEM((NRELP,), f32),
            pltpu.VMEM((EPW,), f32),
            pltpu.VMEM((DRT, D), f32),
            pltpu.VMEM((1, DRT), jnp.int32),
            pltpu.VMEM((GCH, D), f32),
            pltpu.VMEM((GCH, D), f32),
            pltpu.VMEM_SHARED((DRT, D), f32),
            pltpu.SemaphoreType.DMA,
            pltpu.SemaphoreType.DMA,
        ],
    )
    return kern(xu, src, dst, et, a_src, a_dst, a_rel)


def _weight_body(denp_h, dst_h, ex_h, norm_h, w_h,
                 d0_v, d1_v, dst_v, ex_v, norm_v):
    c = lax.axis_index("c")
    s = lax.axis_index("s")
    wid = s * NC + c
    base = wid * EPW
    pltpu.sync_copy(denp_h.at[pl.ds(0, DRT)], d0_v)
    pltpu.sync_copy(denp_h.at[pl.ds(DRT, DRT)], d1_v)
    pltpu.sync_copy(dst_h.at[pl.ds(base, EPW)], dst_v)
    pltpu.sync_copy(ex_h.at[pl.ds(base, EPW)], ex_v)
    pltpu.sync_copy(norm_h.at[pl.ds(base, EPW)], norm_v)

    def _sum(i, _):
        for j in range(D // 16):
            d0_v[i, pl.ds(j * 16, 16)] = (d0_v[i, pl.ds(j * 16, 16)]
                                          + d1_v[i, pl.ds(j * 16, 16)])
        return 0
    lax.fori_loop(0, DRT, _sum, 0)

    def _w(ei, _):
        off = ei * 16
        d16 = dst_v[pl.ds(off, 16)]
        row = jnp.right_shift(d16, 7)
        lane = jnp.bitwise_and(d16, 127)
        den = plsc.load_gather(d0_v, [row, lane])
        ex16 = ex_v[pl.ds(off, 16)]
        nm16 = norm_v[pl.ds(off, 16)]
        ex_v[pl.ds(off, 16)] = ex16 * nm16 / den * (1.0 / 3.0)
        return 0
    lax.fori_loop(0, EPW // 16, _w, 0)
    pltpu.sync_copy(ex_v, w_h.at[pl.ds(base, EPW)])


def _weight_pass(denp, dst, ex, norm):
    f32 = jnp.float32
    mesh = plsc.VectorSubcoreMesh(core_axis_name="c", subcore_axis_name="s")
    kern = pl.kernel(
        _weight_body,
        out_type=jax.ShapeDtypeStruct((E,), f32),
        mesh=mesh,
        compiler_params=pltpu.CompilerParams(
            needs_layout_passes=False, use_tc_tiling_on_sc=False),
        scratch_types=[
            pltpu.VMEM((DRT, D), f32),
            pltpu.VMEM((DRT, D), f32),
            pltpu.VMEM((EPW,), jnp.int32),
            pltpu.VMEM((EPW,), f32),
            pltpu.VMEM((EPW,), f32),
        ],
    )
    return kern(denp, dst, ex, norm)


def _msg_body(u_ref, etm_ref, vhi_ref, vlo_ref, ab_ref, out_ref, v_ref):
    f32 = jnp.float32
    u = u_ref[:]
    dn = (((0,), (0,)), ((), ()))

    def _grp(g, _):
        et_bc = jnp.broadcast_to(etm_ref[0, pl.ds(g, 1), :], (NRELP, 128))
        rows = lax.broadcasted_iota(jnp.int32, (NRELP, 128), 0)
        oht = (et_bc.astype(jnp.int32) == rows).astype(jnp.bfloat16)
        vg = (lax.dot_general(oht, vhi_ref[:], dn,
                              preferred_element_type=f32)
              + lax.dot_general(oht, vlo_ref[:], dn,
                                preferred_element_type=f32))
        v_ref[pl.ds(g * 128, 128), :] = vg
        return 0
    for _g in range(BE // 128):
        _grp(_g, 0)
    v = v_ref[:]
    wcat = jnp.concatenate([u * v[:, :D], u * v[:, D:]], axis=1)
    mp = jnp.dot(wcat, ab_ref[0], preferred_element_type=f32)
    m = jnp.max(mp, axis=1, keepdims=True)
    p = jnp.exp(mp - m)
    ssum = jnp.sum(p, axis=1, keepdims=True)
    out_ref[:] = p / ssum


def _messages(arows, etm, vhi, vlo, ab):
    f32 = jnp.float32
    return pl.pallas_call(
        _msg_body,
        grid=(NB,),
        scratch_shapes=[pltpu.VMEM((BE, 2 * D), jnp.float32)],
        in_specs=[
            pl.BlockSpec((BE, D), lambda i: (i, 0)),
            pl.BlockSpec((1, BE // 128, 128), lambda i: (i, 0, 0)),
            pl.BlockSpec((NRELP, 2 * D), lambda i: (0, 0)),
            pl.BlockSpec((NRELP, 2 * D), lambda i: (0, 0)),
            pl.BlockSpec((1, 2 * D, D), lambda i: (i // (NB // 2), 0, 0)),
        ],
        out_specs=pl.BlockSpec((BE, D), lambda i: (i, 0)),
        out_shape=jax.ShapeDtypeStruct((E, D), f32),
    )(arows, etm, vhi, vlo, ab)


def _scatter_body(msg_h, dst3_h, wt_h, hagg_h, dst_v, w_v, rb0, rb1,
                  shared_h, sem0, sem1):
    c = lax.axis_index("c")
    s = lax.axis_index("s")
    wid = s * NC + c
    base = wid * EPW
    pltpu.sync_copy(dst3_h.at[wid], dst_v)
    pltpu.sync_copy(wt_h.at[pl.ds(base, EPW)], w_v)

    def _zrow(r, _):
        for j in range(D // 16):
            rb0[r, pl.ds(j * 16, 16)] = jnp.zeros((16,), jnp.float32)
        return 0
    lax.fori_loop(0, GCH, _zrow, 0)
    for t in range(NP // NS // GCH):
        pltpu.sync_copy(rb0,
                        shared_h.at[pl.ds(s * (NP // NS) + t * GCH, GCH)])
    plsc.subcore_barrier()

    def _start(ci, rb, sem):
        pltpu.async_copy(msg_h.at[pl.ds(base + ci * GCH, GCH)], rb, sem)

    def _wait(rb, sem):
        pltpu.make_async_copy(msg_h.at[pl.ds(0, GCH)], rb, sem).wait()

    _start(0, rb0, sem0)

    def _scale(rb, ci):
        off = ci * GCH
        for r in range(GCH):
            idxv = jnp.full((16,), off + r, jnp.int32)
            wv = plsc.load_gather(w_v, [idxv])
            for j in range(D // 16):
                rb[r, pl.ds(j * 16, 16)] = rb[r, pl.ds(j * 16, 16)] * wv

    def _chunk(i, _):
        _start(2 * i + 1, rb1, sem1)
        _wait(rb0, sem0)
        _scale(rb0, 2 * i)
        pltpu.sync_copy(rb0, shared_h.at[dst_v.at[2 * i]], add=True)

        @pl.when(i < NCH // 2 - 1)
        def _():
            _start(2 * i + 2, rb0, sem0)
        _wait(rb1, sem1)
        _scale(rb1, 2 * i + 1)
        pltpu.sync_copy(rb1, shared_h.at[dst_v.at[2 * i + 1]], add=True)
        return 0
    lax.fori_loop(0, NCH // 2, _chunk, 0)
    plsc.subcore_barrier()
    pltpu.sync_copy(shared_h.at[pl.ds(s * (NP // NS), NP // NS)],
                    hagg_h.at[pl.ds(c * NP + s * (NP // NS), NP // NS)])


def _scatter(msg, dst3, weight):
    f32 = jnp.float32
    mesh = plsc.VectorSubcoreMesh(core_axis_name="c", subcore_axis_name="s")
    kern = pl.kernel(
        _scatter_body,
        out_type=jax.ShapeDtypeStruct((NC * NP, D), f32),
        mesh=mesh,
        compiler_params=pltpu.CompilerParams(
            needs_layout_passes=False, use_tc_tiling_on_sc=False),
        scratch_types=[
            pltpu.VMEM((NCH, GCH), jnp.int32),
            pltpu.VMEM((EPW,), f32),
            pltpu.VMEM((GCH, D), f32),
            pltpu.VMEM((GCH, D), f32),
            pltpu.VMEM_SHARED((NP, D), f32),
            pltpu.SemaphoreType.DMA,
            pltpu.SemaphoreType.DMA,
        ],
    )
    return kern(msg, dst3, weight)


def _final_body(hagg_ref, loop_ref, bias_ref, gam_ref, bet_ref, out_ref):
    h = (hagg_ref[0:N, :] + hagg_ref[NP:NP + N, :]
         + loop_ref[:] + bias_ref[:])
    mu = jnp.mean(h, axis=0, keepdims=True)
    d = h - mu
    var = jnp.mean(d * d, axis=0, keepdims=True)
    out_ref[:] = d * (gam_ref[:] * lax.rsqrt(var + 1e-5)) + bet_ref[:]


def _finalize(hagg, loop_t, bias, gamma, beta):
    f32 = jnp.float32
    return pl.pallas_call(
        _final_body,
        out_shape=jax.ShapeDtypeStruct((N, D), f32),
    )(hagg, loop_t, bias, gamma, beta)


def kernel(edge_index, x, x_t, x_i, x_n, rel_repr, edge_type, edge_norm,
           in_w, out_w, loop_w, w_rel, loop_rel, attn_w, bias,
           bn_gamma, bn_beta):
    src = edge_index[0]
    dst = edge_index[1]
    relp = jnp.pad(rel_repr, ((0, NRELP - rel_repr.shape[0]), (0, 0)))

    (xu, a_src, a_dst, a_rel, vhi, vlo, ab, loop_t, relout_p) = _prep(
        x, relp, attn_w, loop_rel, loop_w, in_w, out_w, w_rel)

    arows, ex, denp = _edge_pass(
        xu, src, dst, edge_type,
        a_src.reshape(N), a_dst.reshape(N), a_rel.reshape(NRELP))

    weight = _weight_pass(denp, dst, ex, edge_norm)

    msg = _messages(arows, edge_type.astype(jnp.float32).reshape(NB, BE // 128, 128),
                    vhi, vlo, ab)

    hagg = _scatter(msg, dst.reshape(NW, NCH, GCH), weight)

    out = _finalize(hagg, loop_t, bias.reshape(1, D),
                    bn_gamma.reshape(1, D), bn_beta.reshape(1, D))
    return (out, relout_p[:rel_repr.shape[0], :])
